# baseline (speedup 1.0000x reference)
import functools

import jax
import jax.numpy as jnp
import numpy as np
from jax import lax
from jax.experimental import pallas as pl
from jax.experimental.pallas import tpu as pltpu
from jax.experimental.pallas import tpu_sc as plsc

N = 10000
E = 320000
D = 128
DE = 16
DH = D // 2

NC = 2
NS = 16
EH = E // 2
EPT = EH // NS
B = 80
NCH = EPT // B
NP = 10240
ROWS_PER_TILE = NP // NS


def _make_perm():
    out = np.empty(D, np.int32)
    for b0 in range(0, D, 32):
        for k in range(16):
            out[b0 + 2 * k] = b0 + k
            out[b0 + 2 * k + 1] = b0 + 16 + k
    return out


_PERM = _make_perm()
_PMAT = np.zeros((D, D), np.float32)
_PMAT[_PERM, np.arange(D)] = 1.0


def _nf_prep_body(nf, pm, eps, nfp, nfb):
    x = nf[...]
    nfp[...] = (1.0 + eps[0, 0]) * x
    nfb[...] = jnp.dot(x, pm[...],
                       preferred_element_type=jnp.float32).astype(jnp.bfloat16)


def _nf_prep(node_feat, eps):
    BN = 400
    return pl.pallas_call(
        _nf_prep_body,
        grid=(N // BN,),
        in_specs=[
            pl.BlockSpec((BN, D), lambda i: (i, 0)),
            pl.BlockSpec((D, D), lambda i: (0, 0)),
            pl.BlockSpec((1, 1), lambda i: (0, 0)),
        ],
        out_specs=[pl.BlockSpec((BN, D), lambda i: (i, 0)),
                   pl.BlockSpec((BN, D), lambda i: (i, 0))],
        out_shape=[jax.ShapeDtypeStruct((NP, D), jnp.float32),
                   jax.ShapeDtypeStruct((N, D), jnp.bfloat16)],
    )(node_feat, jnp.asarray(_PMAT), eps.reshape(1, 1))


_BE = 4000


def _edge_mlp_body(row_off, ef_hbm, w1, b1, w2, b2, out, ef_v0, ef_v1,
                   sem0, sem1):
    p = pl.program_id(0)
    npairs = pl.num_programs(0)
    base = row_off + p * (2 * _BE)

    def _cp(r0, buf, sem):
        return pltpu.make_async_copy(ef_hbm.at[pl.ds(r0, _BE), :], buf, sem)

    @pl.when(p == 0)
    def _():
        _cp(base, ef_v0, sem0).start()
        _cp(base + _BE, ef_v1, sem1).start()

    def _sub(buf, sem, r0, out_slot):
        _cp(r0, buf, sem).wait()
        h = jnp.maximum(
            jnp.dot(buf[...], w1[...], preferred_element_type=jnp.float32)
            + b1[...], 0.0)
        out[pl.ds(out_slot * _BE, _BE), :] = (
            jnp.dot(h, w2[...], preferred_element_type=jnp.float32)
            + b2[...]).astype(jnp.bfloat16)

    _sub(ef_v0, sem0, base, 0)
    @pl.when(p + 1 < npairs)
    def _():
        _cp(base + 2 * _BE, ef_v0, sem0).start()
    _sub(ef_v1, sem1, base + _BE, 1)
    @pl.when(p + 1 < npairs)
    def _():
        _cp(base + 3 * _BE, ef_v1, sem1).start()


def _edge_mlp(edge_feat, eW1, eb1, eW2p, eb2p, half):
    wspec = lambda r, c: pl.BlockSpec((r, c), lambda i: (0, 0))
    return pl.pallas_call(
        functools.partial(_edge_mlp_body, half * EH),
        grid=(EH // (2 * _BE),),
        in_specs=[
            pl.BlockSpec(memory_space=pl.ANY),
            wspec(DE, 2 * DE), wspec(1, 2 * DE),
            wspec(2 * DE, D), wspec(1, D),
        ],
        out_specs=pl.BlockSpec((2 * _BE, D), lambda i: (i, 0)),
        out_shape=jax.ShapeDtypeStruct((EH, D), jnp.bfloat16),
        scratch_shapes=[
            pltpu.VMEM((_BE, DE), jnp.float32),
            pltpu.VMEM((_BE, DE), jnp.float32),
            pltpu.SemaphoreType.DMA,
            pltpu.SemaphoreType.DMA,
        ],
    )(edge_feat, eW1, eb1.reshape(1, -1), eW2p, eb2p.reshape(1, -1))


def _sc_body(half, nf_hbm, ei_hbm, e2_hbm, nfp_hbm, out_hbm, src_all, dst_all,
             src_v0, dst_v0, eidx_v0, rows_v0, e_v0, msg_v0,
             src_v1, dst_v1, eidx_v1, rows_v1, e_v1, msg_v1,
             acc, sem_g0, sem_e0, sem_g1, sem_e1):
    c = lax.axis_index("c")
    s = lax.axis_index("s")
    base = s * ROWS_PER_TILE

    if half == 0:
        pltpu.sync_copy(
            nfp_hbm.at[pl.ds(base, ROWS_PER_TILE), pl.ds(c * DH, DH)],
            acc.at[pl.ds(base, ROWS_PER_TILE)])
    else:
        def _zrow(r, carry):
            for j in range(DH // 16):
                msg_v0[r, pl.ds(16 * j, 16)] = jnp.zeros((16,), jnp.float32)
            return carry
        lax.fori_loop(0, B, _zrow, 0)
        for k in range(ROWS_PER_TILE // B):
            pltpu.sync_copy(msg_v0, acc.at[pl.ds(base + k * B, B)])
    plsc.subcore_barrier()

    hoff = half * EH
    pltpu.sync_copy(ei_hbm.at[pl.ds(hoff + s * EPT, EPT)], src_all)
    pltpu.sync_copy(ei_hbm.at[pl.ds(E + hoff + s * EPT, EPT)], dst_all)

    ebase = 2 * s * EPT + c
    iota2 = 2 * lax.iota(jnp.int32, 16)

    def _fire(i, src_v, dst_v, eidx_v, rows_v, e_v, msg_v, sem_g, sem_e):
        for j in range(B // 16):
            sl = pl.ds(16 * j, 16)
            src_v[sl] = src_all[pl.ds(i * B + 16 * j, 16)] * 2 + c
            dst_v[sl] = dst_all[pl.ds(i * B + 16 * j, 16)]
            eidx_v[sl] = (ebase + 2 * (i * B + 16 * j)) + iota2
        cp_g = pltpu.async_copy(nf_hbm.at[src_v], rows_v, sem_g)
        cp_e = pltpu.async_copy(e2_hbm.at[eidx_v], e_v, sem_e)
        return cp_g, cp_e

    def _drain_process(src_v, dst_v, eidx_v, rows_v, e_v, msg_v, sem_g, sem_e):
        pltpu.make_async_copy(nf_hbm.at[src_v], rows_v, sem_g).wait()
        pltpu.make_async_copy(e2_hbm.at[eidx_v], e_v, sem_e).wait()

        def _row(r, rc):
            for g in range(DH // 32):
                sl = pl.ds(32 * g, 32)
                ha, hb = plsc.unpack(rows_v[r, sl],
                                     format=plsc.PackFormat.INTERLEAVED)
                ea, eb = plsc.unpack(e_v[r, sl],
                                     format=plsc.PackFormat.INTERLEAVED)
                msg_v[r, pl.ds(32 * g, 16)] = jnp.maximum(ha + ea, 0.0)
                msg_v[r, pl.ds(32 * g + 16, 16)] = jnp.maximum(hb + eb, 0.0)
            return rc
        lax.fori_loop(0, B, _row, 0)
        pltpu.sync_copy(msg_v, acc.at[dst_v], add=True)

    buf0 = (src_v0, dst_v0, eidx_v0, rows_v0, e_v0, msg_v0, sem_g0, sem_e0)
    buf1 = (src_v1, dst_v1, eidx_v1, rows_v1, e_v1, msg_v1, sem_g1, sem_e1)
    _fire(0, *buf0)
    _fire(1, *buf1)

    def _pair(t, carry):
        _drain_process(*buf0)
        _fire(2 * t + 2, *buf0)
        _drain_process(*buf1)
        _fire(2 * t + 3, *buf1)
        return carry
    if NCH % 2 == 0:
        lax.fori_loop(0, (NCH - 2) // 2, _pair, 0)
        _drain_process(*buf0)
        _drain_process(*buf1)
    else:
        lax.fori_loop(0, (NCH - 3) // 2, _pair, 0)
        _drain_process(*buf0)
        _fire(NCH - 1, *buf0)
        _drain_process(*buf1)
        _drain_process(*buf0)

    plsc.subcore_barrier()
    pltpu.sync_copy(acc.at[pl.ds(base, ROWS_PER_TILE)],
                    out_hbm.at[pl.ds(base, ROWS_PER_TILE), pl.ds(c * DH, DH)])


def _sc_aggregate(nfb64, ei_flat, e2, nfp, half):
    mesh = plsc.VectorSubcoreMesh(core_axis_name="c", subcore_axis_name="s")
    f = functools.partial(
        pl.kernel,
        mesh=mesh,
        out_type=jax.ShapeDtypeStruct((NP, D), jnp.float32),
        scratch_types=[
            pltpu.VMEM((EPT,), jnp.int32),
            pltpu.VMEM((EPT,), jnp.int32),
            pltpu.VMEM((B,), jnp.int32),
            pltpu.VMEM((B,), jnp.int32),
            pltpu.VMEM((B,), jnp.int32),
            pltpu.VMEM((B, DH), jnp.bfloat16),
            pltpu.VMEM((B, DH), jnp.bfloat16),
            pltpu.VMEM((B, DH), jnp.float32),
            pltpu.VMEM((B,), jnp.int32),
            pltpu.VMEM((B,), jnp.int32),
            pltpu.VMEM((B,), jnp.int32),
            pltpu.VMEM((B, DH), jnp.bfloat16),
            pltpu.VMEM((B, DH), jnp.bfloat16),
            pltpu.VMEM((B, DH), jnp.float32),
            pltpu.VMEM_SHARED((NP, DH), jnp.float32),
            pltpu.SemaphoreType.DMA,
            pltpu.SemaphoreType.DMA,
            pltpu.SemaphoreType.DMA,
            pltpu.SemaphoreType.DMA,
        ],
        compiler_params=pltpu.CompilerParams(use_tc_tiling_on_sc=False,
                                             needs_layout_passes=False),
    )(functools.partial(_sc_body, half))
    return f(nfb64, ei_flat, e2, nfp)


def _node_mlp_body(pa, pb, w1, b1, w2, b2, out):
    h = pa[...] + pb[...]
    h1 = jnp.maximum(
        jnp.dot(h, w1[...], preferred_element_type=jnp.float32) + b1[...], 0.0)
    out[...] = jnp.dot(h1, w2[...], preferred_element_type=jnp.float32) + b2[...]


def _node_mlp(pa, pb, W1, b1, W2, b2):
    BN = 400
    return pl.pallas_call(
        _node_mlp_body,
        grid=(N // BN,),
        in_specs=[
            pl.BlockSpec((BN, D), lambda i: (i, 0)),
            pl.BlockSpec((BN, D), lambda i: (i, 0)),
            pl.BlockSpec((D, 2 * D), lambda i: (0, 0)),
            pl.BlockSpec((1, 2 * D), lambda i: (0, 0)),
            pl.BlockSpec((2 * D, D), lambda i: (0, 0)),
            pl.BlockSpec((1, D), lambda i: (0, 0)),
        ],
        out_specs=pl.BlockSpec((BN, D), lambda i: (i, 0)),
        out_shape=jax.ShapeDtypeStruct((N, D), jnp.float32),
    )(pa, pb, W1, b1.reshape(1, -1), W2, b2.reshape(1, -1))


def kernel(node_feat, edge_index, edge_feat, W1, b1, W2, b2,
           eW1, eb1, eW2, eb2, eps):
    ei_flat = edge_index.reshape(2 * E)
    perm = jnp.asarray(_PERM)
    eW2p = jnp.take(eW2, perm, axis=1)
    eb2p = jnp.take(eb2, perm)
    nfp, nfb = _nf_prep(node_feat, eps)
    nfb64 = nfb.reshape(2 * N, DH)
    e_a = _edge_mlp(edge_feat, eW1, eb1, eW2p, eb2p, 0)
    e_b = _edge_mlp(edge_feat, eW1, eb1, eW2p, eb2p, 1)
    pa = _sc_aggregate(nfb64, ei_flat, e_a.reshape(2 * EH, DH), nfp, 0)
    pb = _sc_aggregate(nfb64, ei_flat, e_b.reshape(2 * EH, DH), nfp, 1)
    return _node_mlp(pa, pb, W1, b1, W2, b2)


# speedup vs baseline: 2.0023x; 2.0023x over previous
import functools

import jax
import jax.numpy as jnp
from jax import lax
from jax.experimental import pallas as pl
from jax.experimental.pallas import tpu as pltpu
from jax.experimental.pallas import tpu_sc as plsc

N = 10000
E = 320000
D = 128
DE = 16
DH = D // 2

NC = 2
NS = 16
EH = E // 2
EPT = EH // NS
B = 80
NCH = EPT // B
NP = 10240
ROWS_PER_TILE = NP // NS


def _nf_prep_body(nf, eps, nfp):
    nfp[...] = (1.0 + eps[0, 0]) * nf[...]


def _nf_prep(node_feat, eps):
    BN = 400
    return pl.pallas_call(
        _nf_prep_body,
        grid=(N // BN,),
        in_specs=[
            pl.BlockSpec((BN, D), lambda i: (i, 0)),
            pl.BlockSpec((1, 1), lambda i: (0, 0)),
        ],
        out_specs=pl.BlockSpec((BN, D), lambda i: (i, 0)),
        out_shape=jax.ShapeDtypeStruct((NP, D), jnp.float32),
    )(node_feat, eps.reshape(1, 1))


_BE = 4000


def _edge_mlp_body(row_off, ef_hbm, w1, b1, w2, b2, out, ef_v0, ef_v1,
                   sem0, sem1):
    p = pl.program_id(0)
    npairs = pl.num_programs(0)
    base = row_off + p * (2 * _BE)

    def _cp(r0, buf, sem):
        return pltpu.make_async_copy(ef_hbm.at[pl.ds(r0, _BE), :], buf, sem)

    @pl.when(p == 0)
    def _():
        _cp(base, ef_v0, sem0).start()
        _cp(base + _BE, ef_v1, sem1).start()

    def _sub(buf, sem, r0, out_slot):
        _cp(r0, buf, sem).wait()
        h = jnp.maximum(
            jnp.dot(buf[...], w1[...], preferred_element_type=jnp.float32)
            + b1[...], 0.0)
        out[pl.ds(out_slot * _BE, _BE), :] = (
            jnp.dot(h, w2[...], preferred_element_type=jnp.float32) + b2[...])

    _sub(ef_v0, sem0, base, 0)
    @pl.when(p + 1 < npairs)
    def _():
        _cp(base + 2 * _BE, ef_v0, sem0).start()
    _sub(ef_v1, sem1, base + _BE, 1)
    @pl.when(p + 1 < npairs)
    def _():
        _cp(base + 3 * _BE, ef_v1, sem1).start()


def _edge_mlp(edge_feat, eW1, eb1, eW2, eb2, half):
    wspec = lambda r, c: pl.BlockSpec((r, c), lambda i: (0, 0))
    return pl.pallas_call(
        functools.partial(_edge_mlp_body, half * EH),
        grid=(EH // (2 * _BE),),
        in_specs=[
            pl.BlockSpec(memory_space=pl.ANY),
            wspec(DE, 2 * DE), wspec(1, 2 * DE),
            wspec(2 * DE, D), wspec(1, D),
        ],
        out_specs=pl.BlockSpec((2 * _BE, D), lambda i: (i, 0)),
        out_shape=jax.ShapeDtypeStruct((EH, D), jnp.float32),
        scratch_shapes=[
            pltpu.VMEM((_BE, DE), jnp.float32),
            pltpu.VMEM((_BE, DE), jnp.float32),
            pltpu.SemaphoreType.DMA,
            pltpu.SemaphoreType.DMA,
        ],
    )(edge_feat, eW1, eb1.reshape(1, -1), eW2, eb2.reshape(1, -1))


def _sc_body(half, nf_hbm, ei_hbm, e2_hbm, nfp_hbm, out_hbm, src_all, dst_all,
             src_v0, dst_v0, eidx_v0, rows_v0, e_v0,
             src_v1, dst_v1, eidx_v1, rows_v1, e_v1,
             acc, sem_g0, sem_e0, sem_g1, sem_e1, sem_z):
    c = lax.axis_index("c")
    s = lax.axis_index("s")
    base = s * ROWS_PER_TILE

    if half == 0:
        seed = pltpu.async_copy(
            nfp_hbm.at[pl.ds(base, ROWS_PER_TILE), pl.ds(c * DH, DH)],
            acc.at[pl.ds(base, ROWS_PER_TILE)], sem_z)
        seeds = (seed,)
    else:
        def _zrow(r, carry):
            for j in range(DH // 16):
                rows_v0[r, pl.ds(16 * j, 16)] = jnp.zeros((16,), jnp.float32)
            return carry
        lax.fori_loop(0, B, _zrow, 0)
        seeds = tuple(
            pltpu.async_copy(rows_v0, acc.at[pl.ds(base + k * B, B)], sem_z)
            for k in range(ROWS_PER_TILE // B))

    hoff = half * EH
    pltpu.sync_copy(ei_hbm.at[pl.ds(hoff + s * EPT, EPT)], src_all)
    pltpu.sync_copy(ei_hbm.at[pl.ds(E + hoff + s * EPT, EPT)], dst_all)

    for cp in seeds:
        cp.wait()
    plsc.subcore_barrier()

    ebase = 2 * s * EPT + c
    iota2 = 2 * lax.iota(jnp.int32, 16)

    def _fire(i, src_v, dst_v, eidx_v, rows_v, e_v, sem_g, sem_e):
        for j in range(B // 16):
            sl = pl.ds(16 * j, 16)
            src_v[sl] = src_all[pl.ds(i * B + 16 * j, 16)] * 2 + c
            dst_v[sl] = dst_all[pl.ds(i * B + 16 * j, 16)]
            eidx_v[sl] = (ebase + 2 * (i * B + 16 * j)) + iota2
        cp_g = pltpu.async_copy(nf_hbm.at[src_v], rows_v, sem_g)
        cp_e = pltpu.async_copy(e2_hbm.at[eidx_v], e_v, sem_e)
        return cp_g, cp_e

    def _drain_process(src_v, dst_v, eidx_v, rows_v, e_v, sem_g, sem_e):
        pltpu.make_async_copy(nf_hbm.at[src_v], rows_v, sem_g).wait()
        pltpu.make_async_copy(e2_hbm.at[eidx_v], e_v, sem_e).wait()

        def _row(r, rc):
            for j in range(DH // 16):
                sl = pl.ds(16 * j, 16)
                e_v[r, sl] = jnp.maximum(rows_v[r, sl] + e_v[r, sl], 0.0)
            return rc
        lax.fori_loop(0, B, _row, 0)
        pltpu.sync_copy(e_v, acc.at[dst_v], add=True)

    buf0 = (src_v0, dst_v0, eidx_v0, rows_v0, e_v0, sem_g0, sem_e0)
    buf1 = (src_v1, dst_v1, eidx_v1, rows_v1, e_v1, sem_g1, sem_e1)
    _fire(0, *buf0)
    _fire(1, *buf1)

    def _pair(t, carry):
        _drain_process(*buf0)
        _fire(2 * t + 2, *buf0)
        _drain_process(*buf1)
        _fire(2 * t + 3, *buf1)
        return carry
    if NCH % 2 == 0:
        lax.fori_loop(0, (NCH - 2) // 2, _pair, 0)
        _drain_process(*buf0)
        _drain_process(*buf1)
    else:
        lax.fori_loop(0, (NCH - 3) // 2, _pair, 0)
        _drain_process(*buf0)
        _fire(NCH - 1, *buf0)
        _drain_process(*buf1)
        _drain_process(*buf0)

    plsc.subcore_barrier()
    pltpu.sync_copy(acc.at[pl.ds(base, ROWS_PER_TILE)],
                    out_hbm.at[pl.ds(base, ROWS_PER_TILE), pl.ds(c * DH, DH)])


def _sc_aggregate(nf64, ei_flat, e2, nfp, half):
    mesh = plsc.VectorSubcoreMesh(core_axis_name="c", subcore_axis_name="s")
    f = functools.partial(
        pl.kernel,
        mesh=mesh,
        out_type=jax.ShapeDtypeStruct((NP, D), jnp.float32),
        scratch_types=[
            pltpu.VMEM((EPT,), jnp.int32),
            pltpu.VMEM((EPT,), jnp.int32),
            pltpu.VMEM((B,), jnp.int32),
            pltpu.VMEM((B,), jnp.int32),
            pltpu.VMEM((B,), jnp.int32),
            pltpu.VMEM((B, DH), jnp.float32),
            pltpu.VMEM((B, DH), jnp.float32),
            pltpu.VMEM((B,), jnp.int32),
            pltpu.VMEM((B,), jnp.int32),
            pltpu.VMEM((B,), jnp.int32),
            pltpu.VMEM((B, DH), jnp.float32),
            pltpu.VMEM((B, DH), jnp.float32),
            pltpu.VMEM_SHARED((NP, DH), jnp.float32),
            pltpu.SemaphoreType.DMA,
            pltpu.SemaphoreType.DMA,
            pltpu.SemaphoreType.DMA,
            pltpu.SemaphoreType.DMA,
            pltpu.SemaphoreType.DMA,
        ],
        compiler_params=pltpu.CompilerParams(use_tc_tiling_on_sc=False),
    )(functools.partial(_sc_body, half))
    return f(nf64, ei_flat, e2, nfp)


def _node_mlp_body(pa, pb, w1, b1, w2, b2, out):
    h = pa[...] + pb[...]
    h1 = jnp.maximum(
        jnp.dot(h, w1[...], preferred_element_type=jnp.float32) + b1[...], 0.0)
    out[...] = jnp.dot(h1, w2[...], preferred_element_type=jnp.float32) + b2[...]


def _node_mlp(pa, pb, W1, b1, W2, b2):
    BN = 400
    return pl.pallas_call(
        _node_mlp_body,
        grid=(N // BN,),
        in_specs=[
            pl.BlockSpec((BN, D), lambda i: (i, 0)),
            pl.BlockSpec((BN, D), lambda i: (i, 0)),
            pl.BlockSpec((D, 2 * D), lambda i: (0, 0)),
            pl.BlockSpec((1, 2 * D), lambda i: (0, 0)),
            pl.BlockSpec((2 * D, D), lambda i: (0, 0)),
            pl.BlockSpec((1, D), lambda i: (0, 0)),
        ],
        out_specs=pl.BlockSpec((BN, D), lambda i: (i, 0)),
        out_shape=jax.ShapeDtypeStruct((N, D), jnp.float32),
    )(pa, pb, W1, b1.reshape(1, -1), W2, b2.reshape(1, -1))


def kernel(node_feat, edge_index, edge_feat, W1, b1, W2, b2,
           eW1, eb1, eW2, eb2, eps):
    nf64 = node_feat.reshape(2 * N, DH)
    ei_flat = edge_index.reshape(2 * E)
    nfp = _nf_prep(node_feat, eps)
    e_a = _edge_mlp(edge_feat, eW1, eb1, eW2, eb2, 0)
    e_b = _edge_mlp(edge_feat, eW1, eb1, eW2, eb2, 1)
    pa = _sc_aggregate(nf64, ei_flat, e_a.reshape(2 * EH, DH), nfp, 0)
    pb = _sc_aggregate(nf64, ei_flat, e_b.reshape(2 * EH, DH), nfp, 1)
    return _node_mlp(pa, pb, W1, b1, W2, b2)


# speedup vs baseline: 2.0837x; 1.0407x over previous
import functools

import jax
import jax.numpy as jnp
from jax import lax
from jax.experimental import pallas as pl
from jax.experimental.pallas import tpu as pltpu
from jax.experimental.pallas import tpu_sc as plsc

N = 10000
E = 320000
D = 128
DE = 16
DH = D // 2

NC = 2
NS = 16
EH = E // 2
EPT = EH // NS
B = 80
NCH = EPT // B
NP = 10240
ROWS_PER_TILE = NP // NS


_BE = 4000


def _edge_mlp_body(row_off, ef_hbm, w1, b1, w2, b2, out, ef_v0, ef_v1,
                   sem0, sem1):
    p = pl.program_id(0)
    npairs = pl.num_programs(0)
    base = row_off + p * (2 * _BE)

    def _cp(r0, buf, sem):
        return pltpu.make_async_copy(ef_hbm.at[pl.ds(r0, _BE), :], buf, sem)

    @pl.when(p == 0)
    def _():
        _cp(base, ef_v0, sem0).start()
        _cp(base + _BE, ef_v1, sem1).start()

    def _sub(buf, sem, r0, out_slot):
        _cp(r0, buf, sem).wait()
        h = jnp.maximum(
            jnp.dot(buf[...], w1[...], preferred_element_type=jnp.float32)
            + b1[...], 0.0)
        out[pl.ds(out_slot * _BE, _BE), :] = (
            jnp.dot(h, w2[...], preferred_element_type=jnp.float32) + b2[...])

    _sub(ef_v0, sem0, base, 0)
    @pl.when(p + 1 < npairs)
    def _():
        _cp(base + 2 * _BE, ef_v0, sem0).start()
    _sub(ef_v1, sem1, base + _BE, 1)
    @pl.when(p + 1 < npairs)
    def _():
        _cp(base + 3 * _BE, ef_v1, sem1).start()


def _edge_mlp(edge_feat, eW1, eb1, eW2, eb2, half):
    wspec = lambda r, c: pl.BlockSpec((r, c), lambda i: (0, 0))
    return pl.pallas_call(
        functools.partial(_edge_mlp_body, half * EH),
        grid=(EH // (2 * _BE),),
        in_specs=[
            pl.BlockSpec(memory_space=pl.ANY),
            wspec(DE, 2 * DE), wspec(1, 2 * DE),
            wspec(2 * DE, D), wspec(1, D),
        ],
        out_specs=pl.BlockSpec((2 * _BE, D), lambda i: (i, 0)),
        out_shape=jax.ShapeDtypeStruct((EH, D), jnp.float32),
        scratch_shapes=[
            pltpu.VMEM((_BE, DE), jnp.float32),
            pltpu.VMEM((_BE, DE), jnp.float32),
            pltpu.SemaphoreType.DMA,
            pltpu.SemaphoreType.DMA,
        ],
    )(edge_feat, eW1, eb1.reshape(1, -1), eW2, eb2.reshape(1, -1))


def _sc_body(half, nf_hbm, ei_hbm, e2_hbm, out_hbm, src_all, dst_all,
             src_v0, dst_v0, eidx_v0, rows_v0, e_v0,
             src_v1, dst_v1, eidx_v1, rows_v1, e_v1,
             acc, sem_g0, sem_e0, sem_g1, sem_e1):
    c = lax.axis_index("c")
    s = lax.axis_index("s")
    rows_v = rows_v0

    def _zrow(r, carry):
        for j in range(DH // 16):
            rows_v[r, pl.ds(16 * j, 16)] = jnp.zeros((16,), jnp.float32)
        return carry
    lax.fori_loop(0, B, _zrow, 0)
    base = s * ROWS_PER_TILE
    for k in range(ROWS_PER_TILE // B):
        pltpu.sync_copy(rows_v, acc.at[pl.ds(base + k * B, B)])
    plsc.subcore_barrier()

    hoff = half * EH
    pltpu.sync_copy(ei_hbm.at[pl.ds(hoff + s * EPT, EPT)], src_all)
    pltpu.sync_copy(ei_hbm.at[pl.ds(E + hoff + s * EPT, EPT)], dst_all)

    ebase = 2 * s * EPT + c
    iota2 = 2 * lax.iota(jnp.int32, 16)

    def _fire(i, src_v, dst_v, eidx_v, rows_v, e_v, sem_g, sem_e):
        for j in range(B // 16):
            sl = pl.ds(16 * j, 16)
            src_v[sl] = src_all[pl.ds(i * B + 16 * j, 16)] * 2 + c
            dst_v[sl] = dst_all[pl.ds(i * B + 16 * j, 16)]
            eidx_v[sl] = (ebase + 2 * (i * B + 16 * j)) + iota2
        cp_g = pltpu.async_copy(nf_hbm.at[src_v], rows_v, sem_g)
        cp_e = pltpu.async_copy(e2_hbm.at[eidx_v], e_v, sem_e)
        return cp_g, cp_e

    def _drain_process(src_v, dst_v, eidx_v, rows_v, e_v, sem_g, sem_e):
        pltpu.make_async_copy(nf_hbm.at[src_v], rows_v, sem_g).wait()
        pltpu.make_async_copy(e2_hbm.at[eidx_v], e_v, sem_e).wait()

        def _row(r2, rc):
            for rr in range(2):
                for j in range(DH // 16):
                    sl = pl.ds(16 * j, 16)
                    r = 2 * r2 + rr
                    e_v[r, sl] = jnp.maximum(rows_v[r, sl] + e_v[r, sl], 0.0)
            return rc
        lax.fori_loop(0, B // 2, _row, 0)
        pltpu.sync_copy(e_v, acc.at[dst_v], add=True)

    buf0 = (src_v0, dst_v0, eidx_v0, rows_v0, e_v0, sem_g0, sem_e0)
    buf1 = (src_v1, dst_v1, eidx_v1, rows_v1, e_v1, sem_g1, sem_e1)
    _fire(0, *buf0)
    _fire(1, *buf1)

    def _pair(t, carry):
        _drain_process(*buf0)
        _fire(2 * t + 2, *buf0)
        _drain_process(*buf1)
        _fire(2 * t + 3, *buf1)
        return carry
    if NCH % 2 == 0:
        lax.fori_loop(0, (NCH - 2) // 2, _pair, 0)
        _drain_process(*buf0)
        _drain_process(*buf1)
    else:
        lax.fori_loop(0, (NCH - 3) // 2, _pair, 0)
        _drain_process(*buf0)
        _fire(NCH - 1, *buf0)
        _drain_process(*buf1)
        _drain_process(*buf0)

    plsc.subcore_barrier()
    pltpu.sync_copy(acc.at[pl.ds(base, ROWS_PER_TILE)],
                    out_hbm.at[pl.ds(base, ROWS_PER_TILE), pl.ds(c * DH, DH)])


def _sc_aggregate(nf64, ei_flat, e2, half):
    mesh = plsc.VectorSubcoreMesh(core_axis_name="c", subcore_axis_name="s")
    f = functools.partial(
        pl.kernel,
        mesh=mesh,
        out_type=jax.ShapeDtypeStruct((NP, D), jnp.float32),
        scratch_types=[
            pltpu.VMEM((EPT,), jnp.int32),
            pltpu.VMEM((EPT,), jnp.int32),
            pltpu.VMEM((B,), jnp.int32),
            pltpu.VMEM((B,), jnp.int32),
            pltpu.VMEM((B,), jnp.int32),
            pltpu.VMEM((B, DH), jnp.float32),
            pltpu.VMEM((B, DH), jnp.float32),
            pltpu.VMEM((B,), jnp.int32),
            pltpu.VMEM((B,), jnp.int32),
            pltpu.VMEM((B,), jnp.int32),
            pltpu.VMEM((B, DH), jnp.float32),
            pltpu.VMEM((B, DH), jnp.float32),
            pltpu.VMEM_SHARED((NP, DH), jnp.float32),
            pltpu.SemaphoreType.DMA,
            pltpu.SemaphoreType.DMA,
            pltpu.SemaphoreType.DMA,
            pltpu.SemaphoreType.DMA,
        ],
        compiler_params=pltpu.CompilerParams(use_tc_tiling_on_sc=False),
    )(functools.partial(_sc_body, half))
    return f(nf64, ei_flat, e2)


def _node_mlp_body(nf, pa, pb, eps, w1, b1, w2, b2, out):
    h = (1.0 + eps[0, 0]) * nf[...] + pa[...] + pb[...]
    h1 = jnp.maximum(
        jnp.dot(h, w1[...], preferred_element_type=jnp.float32) + b1[...], 0.0)
    out[...] = jnp.dot(h1, w2[...], preferred_element_type=jnp.float32) + b2[...]


def _node_mlp(node_feat, pa, pb, eps, W1, b1, W2, b2):
    BN = 400
    return pl.pallas_call(
        _node_mlp_body,
        grid=(N // BN,),
        in_specs=[
            pl.BlockSpec((BN, D), lambda i: (i, 0)),
            pl.BlockSpec((BN, D), lambda i: (i, 0)),
            pl.BlockSpec((BN, D), lambda i: (i, 0)),
            pl.BlockSpec((1, 1), lambda i: (0, 0)),
            pl.BlockSpec((D, 2 * D), lambda i: (0, 0)),
            pl.BlockSpec((1, 2 * D), lambda i: (0, 0)),
            pl.BlockSpec((2 * D, D), lambda i: (0, 0)),
            pl.BlockSpec((1, D), lambda i: (0, 0)),
        ],
        out_specs=pl.BlockSpec((BN, D), lambda i: (i, 0)),
        out_shape=jax.ShapeDtypeStruct((N, D), jnp.float32),
    )(node_feat, pa, pb, eps.reshape(1, 1), W1, b1.reshape(1, -1), W2,
      b2.reshape(1, -1))


def kernel(node_feat, edge_index, edge_feat, W1, b1, W2, b2,
           eW1, eb1, eW2, eb2, eps):
    nf64 = node_feat.reshape(2 * N, DH)
    ei_flat = edge_index.reshape(2 * E)
    e_a = _edge_mlp(edge_feat, eW1, eb1, eW2, eb2, 0)
    e_b = _edge_mlp(edge_feat, eW1, eb1, eW2, eb2, 1)
    pa = _sc_aggregate(nf64, ei_flat, e_a.reshape(2 * EH, DH), 0)
    pb = _sc_aggregate(nf64, ei_flat, e_b.reshape(2 * EH, DH), 1)
    return _node_mlp(node_feat, pa, pb, eps, W1, b1, W2, b2)
